# Initial kernel scaffold; baseline (speedup 1.0000x reference)
import jax
import jax.numpy as jnp
from jax.experimental import pallas as pl


def kernel(users, seqs, posItems, negItems, embedUser, embedItem):
    raise NotImplementedError("write your pallas kernel here")


# speedup vs baseline: 1.0256x; 1.0256x over previous
import jax
import jax.numpy as jnp
from jax import lax
from jax.experimental import pallas as pl
from jax.experimental.pallas import tpu as pltpu
from jax.experimental.pallas import tpu_sc as plsc

B = 16384
HIST = 50
D = 16
NC = 2
NS = 16
NW = NC * NS
N_PER_W = B // NW
C = 64
NCHUNK = N_PER_W // C
SROWS = C * HIST
SIDX_ROWS = SROWS // 128


def _body(seqs_hbm, users_hbm, pos_hbm, neg_hbm, eu_hbm, ei_hbm, out_hbm,
          sidx, srows, uidx, pidx, nidx, urows, prows, nrows, psc, nsc,
          pt, nt, sem0, sem1):
    wid = lax.axis_index("s") * NC + lax.axis_index("c")
    base = wid * N_PER_W
    sems = (sem0, sem1)
    lane = lax.iota(jnp.int32, 16)

    def fire(c):
        s = c % 2
        g = wid * NCHUNK + c
        pltpu.sync_copy(seqs_hbm.at[pl.ds(g * SROWS, SROWS)], sidx.at[s])
        pltpu.sync_copy(users_hbm.at[pl.ds(g * C, C)], uidx.at[s])
        pltpu.sync_copy(pos_hbm.at[pl.ds(g * C, C)], pidx.at[s])
        pltpu.sync_copy(neg_hbm.at[pl.ds(g * C, C)], nidx.at[s])
        cps = []
        for j in range(SIDX_ROWS):
            cps.append(pltpu.async_copy(
                ei_hbm.at[sidx.at[s].at[pl.ds(j * 128, 128)]],
                srows.at[s].at[pl.ds(j * 128, 128)], sems[s]))
        cps.append(pltpu.async_copy(eu_hbm.at[uidx.at[s]], urows.at[s], sems[s]))
        cps.append(pltpu.async_copy(ei_hbm.at[pidx.at[s]], prows.at[s], sems[s]))
        cps.append(pltpu.async_copy(ei_hbm.at[nidx.at[s]], nrows.at[s], sems[s]))
        return cps

    def compute(c):
        s = c % 2
        srows_s, urows_s, prows_s, nrows_s = (
            srows.at[s], urows.at[s], prows.at[s], nrows.at[s])

        def group(g, _):
            def item(l, _):
                i = g * 16 + l
                ib = i * HIST
                a0 = srows_s[ib + 0, :]
                a1 = srows_s[ib + 1, :]
                a2 = srows_s[ib + 2, :]
                a3 = srows_s[ib + 3, :]
                for j in range(4, HIST, 4):
                    a0 = a0 + srows_s[ib + j + 0, :]
                    a1 = a1 + srows_s[ib + j + 1, :]
                    if j + 2 < HIST:
                        a2 = a2 + srows_s[ib + j + 2, :]
                        a3 = a3 + srows_s[ib + j + 3, :]
                acc = (a0 + a1) + (a2 + a3)
                h = urows_s[i, :] + acc * (1.0 / HIST)
                col = jnp.full((16,), l, jnp.int32)
                plsc.store_scatter(pt, [lane, col], h * prows_s[i, :])
                plsc.store_scatter(nt, [lane, col], h * nrows_s[i, :])
                return 0

            lax.fori_loop(0, 16, item, 0)
            pvec = pt[0, :]
            nvec = nt[0, :]
            for d in range(1, D):
                pvec = pvec + pt[d, :]
                nvec = nvec + nt[d, :]
            off = (c * C) + g * 16
            psc[pl.ds(off, 16)] = pvec
            nsc[pl.ds(off, 16)] = nvec
            return 0

        lax.fori_loop(0, C // 16, group, 0)

    pending = fire(0)
    for c in range(NCHUNK):
        nxt = fire(c + 1) if c + 1 < NCHUNK else []
        for cp in pending:
            cp.wait()
        compute(c)
        pending = nxt

    pltpu.sync_copy(psc, out_hbm.at[pl.ds(base, N_PER_W)])
    pltpu.sync_copy(nsc, out_hbm.at[pl.ds(B + base, N_PER_W)])


@jax.jit
def kernel(users, seqs, posItems, negItems, embedUser, embedItem):
    seqs_r = seqs.reshape(B * HIST)

    mesh = plsc.VectorSubcoreMesh(core_axis_name="c", subcore_axis_name="s")
    run = pl.kernel(
        _body,
        out_type=jax.ShapeDtypeStruct((2 * B,), jnp.float32),
        mesh=mesh,
        compiler_params=pltpu.CompilerParams(
            needs_layout_passes=False, use_tc_tiling_on_sc=False),
        scratch_types=[
            pltpu.VMEM((2, SROWS), jnp.int32),
            pltpu.VMEM((2, SROWS, D), jnp.float32),
            pltpu.VMEM((2, C), jnp.int32),
            pltpu.VMEM((2, C), jnp.int32),
            pltpu.VMEM((2, C), jnp.int32),
            pltpu.VMEM((2, C, D), jnp.float32),
            pltpu.VMEM((2, C, D), jnp.float32),
            pltpu.VMEM((2, C, D), jnp.float32),
            pltpu.VMEM((N_PER_W,), jnp.float32),
            pltpu.VMEM((N_PER_W,), jnp.float32),
            pltpu.VMEM((D, 16), jnp.float32),
            pltpu.VMEM((D, 16), jnp.float32),
            pltpu.SemaphoreType.DMA,
            pltpu.SemaphoreType.DMA,
        ],
    )
    out = run(seqs_r, users, posItems, negItems, embedUser, embedItem)
    return out.reshape(2, B)
